# baseline (speedup 1.0000x reference)
import functools

import numpy as np

import jax
import jax.numpy as jnp
from jax.experimental import pallas as pl
from jax.experimental.pallas import tpu as pltpu

_LANE = 128


def _gap_mxu_kernel(x_ref, m_ref, o_ref, *, inv_hw):
    acc = jnp.dot(x_ref[...], m_ref[...], preferred_element_type=jnp.float32)
    o_ref[...] = (acc * inv_hw).astype(o_ref.dtype)


def _group_mask(hw: int) -> np.ndarray:
    k = hw * _LANE
    m = np.zeros((k, _LANE), np.float32)
    i = np.arange(k)
    m[i, i // hw] = 1.0
    return m


def kernel(x):
    N, C, H, W = x.shape
    hw = H * W
    rows = N * C

    pad_rows = (-rows) % _LANE
    if pad_rows:
        flat = jnp.concatenate(
            [x.reshape(rows * hw), jnp.zeros((pad_rows * hw,), x.dtype)])
    else:
        flat = x.reshape(rows * hw)
    rows_p = rows + pad_rows

    s = rows_p // _LANE
    k = hw * _LANE
    x2 = flat.reshape(s, k)

    ts = s
    for cand in (256, 128, 64, 32, 16, 8, 4, 2, 1):
        if s % cand == 0:
            ts = cand
            break
    n_tiles = s // ts

    m = jnp.asarray(_group_mask(hw))

    out = pl.pallas_call(
        functools.partial(_gap_mxu_kernel, inv_hw=1.0 / float(hw)),
        out_shape=jax.ShapeDtypeStruct((s, _LANE), jnp.float32),
        grid=(n_tiles,),
        in_specs=[
            pl.BlockSpec((ts, k), lambda i: (i, 0)),
            pl.BlockSpec((k, _LANE), lambda i: (0, 0)),
        ],
        out_specs=pl.BlockSpec((ts, _LANE), lambda i: (i, 0)),
        compiler_params=pltpu.CompilerParams(
            dimension_semantics=("parallel",),
            vmem_limit_bytes=64 * 1024 * 1024,
        ),
    )(x2, m)

    out_flat = out.reshape(rows_p)[:rows]
    return out_flat.reshape(N, C).astype(x.dtype)


# speedup vs baseline: 34.1033x; 34.1033x over previous
import functools

import jax
import jax.numpy as jnp
from jax.experimental import pallas as pl
from jax.experimental.pallas import tpu as pltpu


def _gap_kernel(x_ref, o_ref, *, inv_hw):
    acc = jnp.sum(x_ref[...].astype(jnp.float32), axis=0)
    acc = jnp.sum(acc, axis=0)
    o_ref[...] = (acc * inv_hw).astype(o_ref.dtype)


def kernel(x):
    N, C, H, W = x.shape
    xt = jnp.transpose(x, (2, 3, 0, 1))

    bn = N
    for cand in (32, 16, 8, 4, 2, 1):
        if N % cand == 0:
            bn = cand
            break
    n_tiles = N // bn

    out = pl.pallas_call(
        functools.partial(_gap_kernel, inv_hw=1.0 / float(H * W)),
        out_shape=jax.ShapeDtypeStruct((N, C), x.dtype),
        grid=(n_tiles,),
        in_specs=[pl.BlockSpec((H, W, bn, C), lambda i: (0, 0, i, 0))],
        out_specs=pl.BlockSpec((bn, C), lambda i: (i, 0)),
        compiler_params=pltpu.CompilerParams(
            dimension_semantics=("parallel",),
            vmem_limit_bytes=64 * 1024 * 1024,
        ),
    )(xt)
    return out


# speedup vs baseline: 35.6392x; 1.0450x over previous
import functools

import jax
import jax.numpy as jnp
from jax.experimental import pallas as pl
from jax.experimental.pallas import tpu as pltpu


def _gap_kernel(x_ref, o_ref, *, inv_hw):
    acc = jnp.sum(x_ref[...].astype(jnp.float32), axis=0)
    acc = jnp.sum(acc, axis=0)
    o_ref[...] = (acc * inv_hw).astype(o_ref.dtype)


def kernel(x):
    N, C, H, W = x.shape
    xt = jnp.transpose(x, (2, 3, 0, 1))

    bn = N
    for cand in (16, 8, 4, 2, 1):
        if N % cand == 0:
            bn = cand
            break
    n_tiles = N // bn

    out = pl.pallas_call(
        functools.partial(_gap_kernel, inv_hw=1.0 / float(H * W)),
        out_shape=jax.ShapeDtypeStruct((N, C), x.dtype),
        grid=(n_tiles,),
        in_specs=[pl.BlockSpec((H, W, bn, C), lambda i: (0, 0, i, 0))],
        out_specs=pl.BlockSpec((bn, C), lambda i: (i, 0)),
        compiler_params=pltpu.CompilerParams(
            dimension_semantics=("parallel",),
            vmem_limit_bytes=64 * 1024 * 1024,
        ),
    )(xt)
    return out
